# Initial kernel scaffold; baseline (speedup 1.0000x reference)
import jax
import jax.numpy as jnp
from jax.experimental import pallas as pl


def kernel(x, domain_id, W1, b1, W2, b2, W3, b3, loraA1, loraB1, loraA2, loraB2, loraA3, loraB3, dom_emb, layer_pos, Wi1, bi1, gi, bLNi, Wi2, bi2, Wq1, bq1, gq, bLNq, Wq2, bq2, Wt1, bt1, Wt2, bt2):
    raise NotImplementedError("write your pallas kernel here")


# speedup vs baseline: 5.0659x; 5.0659x over previous
import jax
import jax.numpy as jnp
from jax.experimental import pallas as pl
from jax.experimental.pallas import tpu as pltpu

B = 4096
NF = 26
ED = 64
IN = NF * ED
DIMS = [256, 128, 64]
E = 8
R = 16
L = 3
D = 4
KE = 2
KL = 2
DH = 64
LP = 32
SCALING = 1.0

T = 256


def _ln(x, g, b):
    m = jnp.mean(x, axis=-1, keepdims=True)
    v = jnp.mean((x - m) ** 2, axis=-1, keepdims=True)
    return (x - m) * jax.lax.rsqrt(v + 1e-5) * g + b


def _top2_softmax(v):
    m1 = jnp.max(v, axis=-1, keepdims=True)
    neg = jnp.float32(-3.0e38)
    v2 = jnp.where(v >= m1, neg, v)
    m2 = jnp.max(v2, axis=-1, keepdims=True)
    mask = v >= m2
    e = jnp.where(mask, jnp.exp(v - m1), 0.0)
    return e / jnp.sum(e, axis=-1, keepdims=True)


def _body(x_ref, dom_ref, rin_ref,
          wi1_ref, bi1_ref, gi_ref, blni_ref, wi2_ref, bi2_ref,
          wq1_ref, bq1_ref, gq_ref, blnq_ref, wq2_ref, bq2_ref,
          w1_ref, b1_ref, a1_ref, bc1_ref,
          w2_ref, b2_ref, a2_ref, bc2_ref,
          w3_ref, b3_ref, a3_ref, bc3_ref,
          wt1_ref, bt1_ref, wt2_ref, bt2_ref,
          out_ref, gates_ref):
    i = pl.program_id(0)

    @pl.when(i == 0)
    def _compute_gates():
        lane3 = jax.lax.broadcasted_iota(jnp.int32, (D, L), 1)
        inter = jnp.zeros((D, L), jnp.float32)
        for l in range(L):
            r = rin_ref[l]
            zi = jnp.dot(r, wi1_ref[...], preferred_element_type=jnp.float32) + bi1_ref[...]
            hi = jnp.maximum(_ln(zi, gi_ref[...], blni_ref[...]), 0.0)
            il = jnp.dot(hi, wi2_ref[...], preferred_element_type=jnp.float32) + bi2_ref[...]
            inter = jnp.where(lane3 == l, il, inter)
            zq = jnp.dot(r, wq1_ref[...], preferred_element_type=jnp.float32) + bq1_ref[...]
            hq = jnp.maximum(_ln(zq, gq_ref[...], blnq_ref[...]), 0.0)
            ql = jnp.dot(hq, wq2_ref[...], preferred_element_type=jnp.float32) + bq2_ref[...]
            gates_ref[l] = _top2_softmax(ql)
        layer_w = _top2_softmax(inter)
        for l in range(L):
            lw = jnp.sum(jnp.where(lane3 == l, layer_w, 0.0), axis=-1, keepdims=True)
            gates_ref[l] = gates_ref[l] * lw * SCALING

    dom = dom_ref[...]
    oh = (jax.lax.broadcasted_iota(jnp.int32, (T, D), 1) == dom).astype(jnp.float32)
    rep = (jax.lax.broadcasted_iota(jnp.int32, (E, E * R), 0)
           == jax.lax.broadcasted_iota(jnp.int32, (E, E * R), 1) // R).astype(jnp.float32)

    h = x_ref[...]
    layers = ((w1_ref, b1_ref, a1_ref, bc1_ref),
              (w2_ref, b2_ref, a2_ref, bc2_ref),
              (w3_ref, b3_ref, a3_ref, bc3_ref))
    for l, (wr, br, ar, bcr) in enumerate(layers):
        z = jnp.dot(h, wr[...], preferred_element_type=jnp.float32) + br[...]
        a = jnp.dot(h, ar[...], preferred_element_type=jnp.float32)
        g = jnp.dot(oh, gates_ref[l], preferred_element_type=jnp.float32)
        s = jnp.dot(g, rep, preferred_element_type=jnp.float32)
        lora = jnp.dot(a * s, bcr[...], preferred_element_type=jnp.float32)
        h = jnp.maximum(z + lora, 0.0)

    acc = jnp.zeros((T, 8), jnp.float32)
    for d in range(D):
        sel = (dom == d).astype(jnp.float32)
        td = jnp.dot(h, wt1_ref[d], preferred_element_type=jnp.float32) + bt1_ref[d]
        acc = acc + sel * td
    t = jnp.maximum(acc, 0.0)
    acc2 = jnp.zeros((T, 1), jnp.float32)
    for d in range(D):
        sel = (dom == d).astype(jnp.float32)
        od = jnp.dot(t, wt2_ref[d], preferred_element_type=jnp.float32) + bt2_ref[d]
        acc2 = acc2 + sel * od
    out_ref[...] = acc2


def kernel(x, domain_id, W1, b1, W2, b2, W3, b3, loraA1, loraB1, loraA2, loraB2,
           loraA3, loraB3, dom_emb, layer_pos, Wi1, bi1, gi, bLNi, Wi2, bi2,
           Wq1, bq1, gq, bLNq, Wq2, bq2, Wt1, bt1, Wt2, bt2):
    dom2d = domain_id.astype(jnp.int32).reshape(B, 1)
    rin = jnp.concatenate([
        jnp.broadcast_to(dom_emb[None, :, :], (L, D, DH)),
        jnp.broadcast_to(layer_pos[:, None, :], (L, D, LP)),
    ], axis=-1)

    dims = [IN] + DIMS
    WTs = [W1.T, W2.T, W3.T]
    ATs = [loraA1.reshape(E * R, dims[0]).T,
           loraA2.reshape(E * R, dims[1]).T,
           loraA3.reshape(E * R, dims[2]).T]
    BCs = [jnp.swapaxes(loraB1, 1, 2).reshape(E * R, dims[1]),
           jnp.swapaxes(loraB2, 1, 2).reshape(E * R, dims[2]),
           jnp.swapaxes(loraB3, 1, 2).reshape(E * R, dims[3])]
    brs = [b1.reshape(1, -1), b2.reshape(1, -1), b3.reshape(1, -1)]

    Wt1T = jnp.swapaxes(Wt1, 1, 2)
    bt1r = bt1[:, None, :]
    Wt2T = jnp.swapaxes(Wt2, 1, 2)
    bt2r = bt2[:, None, :]

    full = lambda shape: pl.BlockSpec(shape, lambda i: tuple(0 for _ in shape))
    grid = B // T
    out = pl.pallas_call(
        _body,
        grid=(grid,),
        in_specs=[
            pl.BlockSpec((T, IN), lambda i: (i, 0)),
            pl.BlockSpec((T, 1), lambda i: (i, 0)),
            full((L, D, DH + LP)),
            full((DH + LP, 64)), full((1, 64)), full((1, 64)), full((1, 64)),
            full((64, 1)), full((1, 1)),
            full((DH + LP, 64)), full((1, 64)), full((1, 64)), full((1, 64)),
            full((64, E)), full((1, E)),
            full((dims[0], dims[1])), full((1, dims[1])),
            full((dims[0], E * R)), full((E * R, dims[1])),
            full((dims[1], dims[2])), full((1, dims[2])),
            full((dims[1], E * R)), full((E * R, dims[2])),
            full((dims[2], dims[3])), full((1, dims[3])),
            full((dims[2], E * R)), full((E * R, dims[3])),
            full((D, dims[3], 8)), full((D, 1, 8)),
            full((D, 8, 1)), full((D, 1, 1)),
        ],
        out_specs=pl.BlockSpec((T, 1), lambda i: (i, 0)),
        out_shape=jax.ShapeDtypeStruct((B, 1), jnp.float32),
        scratch_shapes=[pltpu.VMEM((L, D, E), jnp.float32)],
    )(x, dom2d, rin,
      Wi1.T, bi1.reshape(1, -1), gi.reshape(1, -1), bLNi.reshape(1, -1),
      Wi2.T, bi2.reshape(1, -1),
      Wq1.T, bq1.reshape(1, -1), gq.reshape(1, -1), bLNq.reshape(1, -1),
      Wq2.T, bq2.reshape(1, -1),
      WTs[0], brs[0], ATs[0], BCs[0],
      WTs[1], brs[1], ATs[1], BCs[1],
      WTs[2], brs[2], ATs[2], BCs[2],
      Wt1T, bt1r, Wt2T, bt2r)
    return out


# speedup vs baseline: 5.2480x; 1.0359x over previous
import jax
import jax.numpy as jnp
from jax.experimental import pallas as pl
from jax.experimental.pallas import tpu as pltpu

B = 4096
NF = 26
ED = 64
IN = NF * ED
DIMS = [256, 128, 64]
E = 8
R = 16
L = 3
D = 4
KE = 2
KL = 2
DH = 64
LP = 32
SCALING = 1.0

T = 256


def _ln(x, g, b):
    m = jnp.mean(x, axis=-1, keepdims=True)
    v = jnp.mean((x - m) ** 2, axis=-1, keepdims=True)
    return (x - m) * jax.lax.rsqrt(v + 1e-5) * g + b


def _top2_softmax(v):
    m1 = jnp.max(v, axis=-1, keepdims=True)
    neg = jnp.float32(-3.0e38)
    v2 = jnp.where(v >= m1, neg, v)
    m2 = jnp.max(v2, axis=-1, keepdims=True)
    mask = v >= m2
    e = jnp.where(mask, jnp.exp(v - m1), 0.0)
    return e / jnp.sum(e, axis=-1, keepdims=True)


def _body(x_ref, dom_ref, rin_ref,
          wi1_ref, bi1_ref, gi_ref, blni_ref, wi2_ref, bi2_ref,
          wq1_ref, bq1_ref, gq_ref, blnq_ref, wq2_ref, bq2_ref,
          w1_ref, b1_ref, bc1_ref,
          w2_ref, b2_ref, bc2_ref,
          w3_ref, b3_ref, bc3_ref,
          wt1_ref, bt1_ref, wt2_ref, bt2_ref,
          out_ref, gates_ref):
    i = pl.program_id(0)

    @pl.when(i == 0)
    def _compute_gates():
        lane3 = jax.lax.broadcasted_iota(jnp.int32, (D, L), 1)
        inter = jnp.zeros((D, L), jnp.float32)
        for l in range(L):
            r = rin_ref[l]
            zi = jnp.dot(r, wi1_ref[...], preferred_element_type=jnp.float32) + bi1_ref[...]
            hi = jnp.maximum(_ln(zi, gi_ref[...], blni_ref[...]), 0.0)
            il = jnp.dot(hi, wi2_ref[...], preferred_element_type=jnp.float32) + bi2_ref[...]
            inter = jnp.where(lane3 == l, il, inter)
            zq = jnp.dot(r, wq1_ref[...], preferred_element_type=jnp.float32) + bq1_ref[...]
            hq = jnp.maximum(_ln(zq, gq_ref[...], blnq_ref[...]), 0.0)
            ql = jnp.dot(hq, wq2_ref[...], preferred_element_type=jnp.float32) + bq2_ref[...]
            gates_ref[l] = _top2_softmax(ql)
        layer_w = _top2_softmax(inter)
        for l in range(L):
            lw = jnp.sum(jnp.where(lane3 == l, layer_w, 0.0), axis=-1, keepdims=True)
            gates_ref[l] = gates_ref[l] * lw * SCALING

    dom = dom_ref[...]
    oh = (jax.lax.broadcasted_iota(jnp.int32, (T, D), 1) == dom).astype(jnp.float32)
    rep = (jax.lax.broadcasted_iota(jnp.int32, (E, E * R), 0)
           == jax.lax.broadcasted_iota(jnp.int32, (E, E * R), 1) // R).astype(jnp.float32)

    h = x_ref[...].astype(jnp.bfloat16)
    layers = ((w1_ref, b1_ref, bc1_ref, DIMS[0]),
              (w2_ref, b2_ref, bc2_ref, DIMS[1]),
              (w3_ref, b3_ref, bc3_ref, DIMS[2]))
    for l, (wa, br, bcr, out_d) in enumerate(layers):
        za = jnp.dot(h, wa[...], preferred_element_type=jnp.float32)
        a = za[:, :E * R]
        z = za[:, E * R:] + br[...]
        g = jnp.dot(oh, gates_ref[l], preferred_element_type=jnp.float32)
        s = jnp.dot(g, rep, preferred_element_type=jnp.float32)
        lora = jnp.dot((a * s).astype(jnp.bfloat16), bcr[...],
                       preferred_element_type=jnp.float32)
        hf = jnp.maximum(z + lora, 0.0)
        h = hf.astype(jnp.bfloat16)
    h = hf

    acc = jnp.zeros((T, 8), jnp.float32)
    for d in range(D):
        sel = (dom == d).astype(jnp.float32)
        td = jnp.dot(h, wt1_ref[d], preferred_element_type=jnp.float32) + bt1_ref[d]
        acc = acc + sel * td
    t = jnp.maximum(acc, 0.0)
    acc2 = jnp.zeros((T, 1), jnp.float32)
    for d in range(D):
        sel = (dom == d).astype(jnp.float32)
        od = jnp.dot(t, wt2_ref[d], preferred_element_type=jnp.float32) + bt2_ref[d]
        acc2 = acc2 + sel * od
    out_ref[...] = acc2


def kernel(x, domain_id, W1, b1, W2, b2, W3, b3, loraA1, loraB1, loraA2, loraB2,
           loraA3, loraB3, dom_emb, layer_pos, Wi1, bi1, gi, bLNi, Wi2, bi2,
           Wq1, bq1, gq, bLNq, Wq2, bq2, Wt1, bt1, Wt2, bt2):
    dom2d = domain_id.astype(jnp.int32).reshape(B, 1)
    rin = jnp.concatenate([
        jnp.broadcast_to(dom_emb[None, :, :], (L, D, DH)),
        jnp.broadcast_to(layer_pos[:, None, :], (L, D, LP)),
    ], axis=-1)

    dims = [IN] + DIMS
    bf = jnp.bfloat16
    WAs = [jnp.concatenate([loraA1.reshape(E * R, dims[0]).T, W1.T], axis=1).astype(bf),
           jnp.concatenate([loraA2.reshape(E * R, dims[1]).T, W2.T], axis=1).astype(bf),
           jnp.concatenate([loraA3.reshape(E * R, dims[2]).T, W3.T], axis=1).astype(bf)]
    BCs = [jnp.swapaxes(loraB1, 1, 2).reshape(E * R, dims[1]).astype(bf),
           jnp.swapaxes(loraB2, 1, 2).reshape(E * R, dims[2]).astype(bf),
           jnp.swapaxes(loraB3, 1, 2).reshape(E * R, dims[3]).astype(bf)]
    brs = [b1.reshape(1, -1), b2.reshape(1, -1), b3.reshape(1, -1)]

    Wt1T = jnp.swapaxes(Wt1, 1, 2)
    bt1r = bt1[:, None, :]
    Wt2T = jnp.swapaxes(Wt2, 1, 2)
    bt2r = bt2[:, None, :]

    full = lambda shape: pl.BlockSpec(shape, lambda i: tuple(0 for _ in shape))
    grid = B // T
    out = pl.pallas_call(
        _body,
        grid=(grid,),
        in_specs=[
            pl.BlockSpec((T, IN), lambda i: (i, 0)),
            pl.BlockSpec((T, 1), lambda i: (i, 0)),
            full((L, D, DH + LP)),
            full((DH + LP, 64)), full((1, 64)), full((1, 64)), full((1, 64)),
            full((64, 1)), full((1, 1)),
            full((DH + LP, 64)), full((1, 64)), full((1, 64)), full((1, 64)),
            full((64, E)), full((1, E)),
            full((dims[0], E * R + dims[1])), full((1, dims[1])), full((E * R, dims[1])),
            full((dims[1], E * R + dims[2])), full((1, dims[2])), full((E * R, dims[2])),
            full((dims[2], E * R + dims[3])), full((1, dims[3])), full((E * R, dims[3])),
            full((D, dims[3], 8)), full((D, 1, 8)),
            full((D, 8, 1)), full((D, 1, 1)),
        ],
        out_specs=pl.BlockSpec((T, 1), lambda i: (i, 0)),
        out_shape=jax.ShapeDtypeStruct((B, 1), jnp.float32),
        scratch_shapes=[pltpu.VMEM((L, D, E), jnp.float32)],
    )(x, dom2d, rin,
      Wi1.T, bi1.reshape(1, -1), gi.reshape(1, -1), bLNi.reshape(1, -1),
      Wi2.T, bi2.reshape(1, -1),
      Wq1.T, bq1.reshape(1, -1), gq.reshape(1, -1), bLNq.reshape(1, -1),
      Wq2.T, bq2.reshape(1, -1),
      WAs[0], brs[0], BCs[0],
      WAs[1], brs[1], BCs[1],
      WAs[2], brs[2], BCs[2],
      Wt1T, bt1r, Wt2T, bt2r)
    return out


# speedup vs baseline: 6.4008x; 1.2197x over previous
import jax
import jax.numpy as jnp
from jax.experimental import pallas as pl
from jax.experimental.pallas import tpu as pltpu

B = 4096
NF = 26
ED = 64
IN = NF * ED
DIMS = [256, 128, 64]
E = 8
R = 16
L = 3
D = 4
KE = 2
KL = 2
DH = 64
LP = 32
SCALING = 1.0

T = 512


def _ln(x, g, b):
    m = jnp.mean(x, axis=-1, keepdims=True)
    v = jnp.mean((x - m) ** 2, axis=-1, keepdims=True)
    return (x - m) * jax.lax.rsqrt(v + 1e-5) * g + b


def _top2_softmax(v):
    m1 = jnp.max(v, axis=-1, keepdims=True)
    neg = jnp.float32(-3.0e38)
    v2 = jnp.where(v >= m1, neg, v)
    m2 = jnp.max(v2, axis=-1, keepdims=True)
    mask = v >= m2
    e = jnp.where(mask, jnp.exp(v - m1), 0.0)
    return e / jnp.sum(e, axis=-1, keepdims=True)


def _body(x_ref, dom_ref, rin_ref,
          wi1_ref, bi1_ref, gi_ref, blni_ref, wi2_ref, bi2_ref,
          wq1_ref, bq1_ref, gq_ref, blnq_ref, wq2_ref, bq2_ref,
          w1_ref, b1_ref, bc1_ref,
          w2_ref, b2_ref, bc2_ref,
          w3_ref, b3_ref, bc3_ref,
          wt1_ref, bt1_ref, wt2_ref, bt2_ref,
          out_ref, stab_ref):
    i = pl.program_id(0)

    @pl.when(i == 0)
    def _compute_gates():
        lane3 = jax.lax.broadcasted_iota(jnp.int32, (D, L), 1)
        rep = (jax.lax.broadcasted_iota(jnp.int32, (E, E * R), 0)
               == jax.lax.broadcasted_iota(jnp.int32, (E, E * R), 1) // R
               ).astype(jnp.float32)
        inter = jnp.zeros((D, L), jnp.float32)
        intra = []
        for l in range(L):
            r = rin_ref[l]
            zi = jnp.dot(r, wi1_ref[...], preferred_element_type=jnp.float32) + bi1_ref[...]
            hi = jnp.maximum(_ln(zi, gi_ref[...], blni_ref[...]), 0.0)
            il = jnp.dot(hi, wi2_ref[...], preferred_element_type=jnp.float32) + bi2_ref[...]
            inter = jnp.where(lane3 == l, il, inter)
            zq = jnp.dot(r, wq1_ref[...], preferred_element_type=jnp.float32) + bq1_ref[...]
            hq = jnp.maximum(_ln(zq, gq_ref[...], blnq_ref[...]), 0.0)
            ql = jnp.dot(hq, wq2_ref[...], preferred_element_type=jnp.float32) + bq2_ref[...]
            intra.append(_top2_softmax(ql))
        layer_w = _top2_softmax(inter)
        for l in range(L):
            lw = jnp.sum(jnp.where(lane3 == l, layer_w, 0.0), axis=-1, keepdims=True)
            gl = intra[l] * lw * SCALING
            stab_ref[l] = jnp.dot(gl, rep, preferred_element_type=jnp.float32)

    dom = dom_ref[...]

    h = x_ref[...].astype(jnp.bfloat16)
    layers = ((w1_ref, b1_ref, bc1_ref, DIMS[0]),
              (w2_ref, b2_ref, bc2_ref, DIMS[1]),
              (w3_ref, b3_ref, bc3_ref, DIMS[2]))
    for l, (wa, br, bcr, out_d) in enumerate(layers):
        za = jnp.dot(h, wa[...], preferred_element_type=jnp.float32)
        a = za[:, :E * R]
        z = za[:, E * R:] + br[...]
        st = stab_ref[l]
        s = jnp.zeros((T, E * R), jnp.float32)
        for d in range(D):
            s = jnp.where(dom == d, st[d:d + 1, :], s)
        lora = jnp.dot((a * s).astype(jnp.bfloat16), bcr[...],
                       preferred_element_type=jnp.float32)
        hf = jnp.maximum(z + lora, 0.0)
        h = hf.astype(jnp.bfloat16)
    h = hf

    t = jnp.maximum(jnp.dot(h, wt1_ref[...], preferred_element_type=jnp.float32)
                    + bt1_ref[...], 0.0)
    blk = jax.lax.broadcasted_iota(jnp.int32, (T, D * 8), 1) // 8
    tm = jnp.where(blk == dom, t, 0.0)
    o = jnp.dot(tm, wt2_ref[...], preferred_element_type=jnp.float32)
    ob = jnp.zeros((T, 1), jnp.float32)
    for d in range(D):
        ob = jnp.where(dom == d, bt2_ref[d:d + 1, :], ob)
    out_ref[...] = o + ob


def kernel(x, domain_id, W1, b1, W2, b2, W3, b3, loraA1, loraB1, loraA2, loraB2,
           loraA3, loraB3, dom_emb, layer_pos, Wi1, bi1, gi, bLNi, Wi2, bi2,
           Wq1, bq1, gq, bLNq, Wq2, bq2, Wt1, bt1, Wt2, bt2):
    dom2d = domain_id.astype(jnp.int32).reshape(B, 1)
    rin = jnp.concatenate([
        jnp.broadcast_to(dom_emb[None, :, :], (L, D, DH)),
        jnp.broadcast_to(layer_pos[:, None, :], (L, D, LP)),
    ], axis=-1)

    dims = [IN] + DIMS
    bf = jnp.bfloat16
    WAs = [jnp.concatenate([loraA1.reshape(E * R, dims[0]).T, W1.T], axis=1).astype(bf),
           jnp.concatenate([loraA2.reshape(E * R, dims[1]).T, W2.T], axis=1).astype(bf),
           jnp.concatenate([loraA3.reshape(E * R, dims[2]).T, W3.T], axis=1).astype(bf)]
    BCs = [jnp.swapaxes(loraB1, 1, 2).reshape(E * R, dims[1]).astype(bf),
           jnp.swapaxes(loraB2, 1, 2).reshape(E * R, dims[2]).astype(bf),
           jnp.swapaxes(loraB3, 1, 2).reshape(E * R, dims[3]).astype(bf)]
    brs = [b1.reshape(1, -1), b2.reshape(1, -1), b3.reshape(1, -1)]

    Wt1all = jnp.transpose(Wt1, (2, 0, 1)).reshape(dims[3], D * 8)
    bt1r = bt1.reshape(1, D * 8)
    Wt2cat = jnp.transpose(Wt2, (0, 2, 1)).reshape(D * 8, 1)
    bt2r = bt2.reshape(D, 1)

    full = lambda shape: pl.BlockSpec(shape, lambda i: tuple(0 for _ in shape))
    grid = B // T
    out = pl.pallas_call(
        _body,
        grid=(grid,),
        in_specs=[
            pl.BlockSpec((T, IN), lambda i: (i, 0)),
            pl.BlockSpec((T, 1), lambda i: (i, 0)),
            full((L, D, DH + LP)),
            full((DH + LP, 64)), full((1, 64)), full((1, 64)), full((1, 64)),
            full((64, 1)), full((1, 1)),
            full((DH + LP, 64)), full((1, 64)), full((1, 64)), full((1, 64)),
            full((64, E)), full((1, E)),
            full((dims[0], E * R + dims[1])), full((1, dims[1])), full((E * R, dims[1])),
            full((dims[1], E * R + dims[2])), full((1, dims[2])), full((E * R, dims[2])),
            full((dims[2], E * R + dims[3])), full((1, dims[3])), full((E * R, dims[3])),
            full((dims[3], D * 8)), full((1, D * 8)),
            full((D * 8, 1)), full((D, 1)),
        ],
        out_specs=pl.BlockSpec((T, 1), lambda i: (i, 0)),
        out_shape=jax.ShapeDtypeStruct((B, 1), jnp.float32),
        scratch_shapes=[pltpu.VMEM((L, D, E * R), jnp.float32)],
    )(x, dom2d, rin,
      Wi1.T, bi1.reshape(1, -1), gi.reshape(1, -1), bLNi.reshape(1, -1),
      Wi2.T, bi2.reshape(1, -1),
      Wq1.T, bq1.reshape(1, -1), gq.reshape(1, -1), bLNq.reshape(1, -1),
      Wq2.T, bq2.reshape(1, -1),
      WAs[0], brs[0], BCs[0],
      WAs[1], brs[1], BCs[1],
      WAs[2], brs[2], BCs[2],
      Wt1all, bt1r, Wt2cat, bt2r)
    return out


# speedup vs baseline: 7.2627x; 1.1347x over previous
import jax
import jax.numpy as jnp
from jax.experimental import pallas as pl
from jax.experimental.pallas import tpu as pltpu

B = 4096
NF = 26
ED = 64
IN = NF * ED
DIMS = [256, 128, 64]
E = 8
R = 16
L = 3
D = 4
KE = 2
KL = 2
DH = 64
LP = 32
SCALING = 1.0

T = 512


def _ln(x, g, b):
    m = jnp.mean(x, axis=-1, keepdims=True)
    v = jnp.mean((x - m) ** 2, axis=-1, keepdims=True)
    return (x - m) * jax.lax.rsqrt(v + 1e-5) * g + b


def _top2_softmax(v):
    m1 = jnp.max(v, axis=-1, keepdims=True)
    neg = jnp.float32(-3.0e38)
    v2 = jnp.where(v >= m1, neg, v)
    m2 = jnp.max(v2, axis=-1, keepdims=True)
    mask = v >= m2
    e = jnp.where(mask, jnp.exp(v - m1), 0.0)
    return e / jnp.sum(e, axis=-1, keepdims=True)


def _body(x_ref, dom_ref, rin_ref,
          wi1_ref, bi1_ref, gi_ref, blni_ref, wi2_ref, bi2_ref,
          wq1_ref, bq1_ref, gq_ref, blnq_ref, wq2_ref, bq2_ref,
          w1_ref, b1_ref, bc1_ref,
          w2_ref, b2_ref, bc2_ref,
          w3_ref, b3_ref, bc3_ref,
          wt1_ref, bt1_ref, wt2_ref, bt2_ref,
          out_ref, stab_ref):
    i = pl.program_id(0)

    @pl.when(i == 0)
    def _compute_gates():
        lane3 = jax.lax.broadcasted_iota(jnp.int32, (D, L), 1)
        rep = (jax.lax.broadcasted_iota(jnp.int32, (E, E * R), 0)
               == jax.lax.broadcasted_iota(jnp.int32, (E, E * R), 1) // R
               ).astype(jnp.float32)
        dn = (((1,), (1,)), ((), ()))
        inter = jnp.zeros((D, L), jnp.float32)
        intra = []
        for l in range(L):
            r = rin_ref[l]
            zi = jax.lax.dot_general(r, wi1_ref[...], dn, preferred_element_type=jnp.float32) + bi1_ref[...]
            hi = jnp.maximum(_ln(zi, gi_ref[...], blni_ref[...]), 0.0)
            il = jnp.sum(hi * wi2_ref[...], axis=-1, keepdims=True) + bi2_ref[...]
            inter = jnp.where(lane3 == l, il, inter)
            zq = jax.lax.dot_general(r, wq1_ref[...], dn, preferred_element_type=jnp.float32) + bq1_ref[...]
            hq = jnp.maximum(_ln(zq, gq_ref[...], blnq_ref[...]), 0.0)
            ql = jax.lax.dot_general(hq, wq2_ref[...], dn, preferred_element_type=jnp.float32) + bq2_ref[...]
            intra.append(_top2_softmax(ql))
        layer_w = _top2_softmax(inter)
        for l in range(L):
            lw = jnp.sum(jnp.where(lane3 == l, layer_w, 0.0), axis=-1, keepdims=True)
            gl = intra[l] * lw * SCALING
            stab_ref[l] = jnp.dot(gl, rep, preferred_element_type=jnp.float32)

    dom = dom_ref[...]

    h = x_ref[...].astype(jnp.bfloat16)
    layers = ((w1_ref, b1_ref, bc1_ref, DIMS[0]),
              (w2_ref, b2_ref, bc2_ref, DIMS[1]),
              (w3_ref, b3_ref, bc3_ref, DIMS[2]))
    for l, (wa, br, bcr, out_d) in enumerate(layers):
        za = jax.lax.dot_general(h, wa[...], (((1,), (1,)), ((), ())),
                                 preferred_element_type=jnp.float32)
        a = za[:, :E * R]
        z = za[:, E * R:] + br[...]
        st = stab_ref[l]
        s = jnp.zeros((T, E * R), jnp.float32)
        for d in range(D):
            s = jnp.where(dom == d, st[d:d + 1, :], s)
        lora = jnp.dot((a * s).astype(jnp.bfloat16), bcr[...],
                       preferred_element_type=jnp.float32)
        hf = jnp.maximum(z + lora, 0.0)
        h = hf.astype(jnp.bfloat16)
    h = hf

    t = jnp.maximum(jnp.dot(h, wt1_ref[...], preferred_element_type=jnp.float32)
                    + bt1_ref[...], 0.0)
    blk = jax.lax.broadcasted_iota(jnp.int32, (T, D * 8), 1) // 8
    tm = jnp.where(blk == dom, t, 0.0)
    o = jnp.dot(tm, wt2_ref[...], preferred_element_type=jnp.float32)
    ob = jnp.zeros((T, 1), jnp.float32)
    for d in range(D):
        ob = jnp.where(dom == d, bt2_ref[d:d + 1, :], ob)
    out_ref[...] = o + ob


def kernel(x, domain_id, W1, b1, W2, b2, W3, b3, loraA1, loraB1, loraA2, loraB2,
           loraA3, loraB3, dom_emb, layer_pos, Wi1, bi1, gi, bLNi, Wi2, bi2,
           Wq1, bq1, gq, bLNq, Wq2, bq2, Wt1, bt1, Wt2, bt2):
    dom2d = domain_id.astype(jnp.int32).reshape(B, 1)
    rin = jnp.concatenate([
        jnp.broadcast_to(dom_emb[None, :, :], (L, D, DH)),
        jnp.broadcast_to(layer_pos[:, None, :], (L, D, LP)),
    ], axis=-1)

    dims = [IN] + DIMS
    bf = jnp.bfloat16
    WAs = [jnp.concatenate([loraA1.reshape(E * R, dims[0]), W1], axis=0).astype(bf),
           jnp.concatenate([loraA2.reshape(E * R, dims[1]), W2], axis=0).astype(bf),
           jnp.concatenate([loraA3.reshape(E * R, dims[2]), W3], axis=0).astype(bf)]
    BCs = [jnp.swapaxes(loraB1, 1, 2).reshape(E * R, dims[1]).astype(bf),
           jnp.swapaxes(loraB2, 1, 2).reshape(E * R, dims[2]).astype(bf),
           jnp.swapaxes(loraB3, 1, 2).reshape(E * R, dims[3]).astype(bf)]
    brs = [b1.reshape(1, -1), b2.reshape(1, -1), b3.reshape(1, -1)]

    Wt1all = jnp.transpose(Wt1, (2, 0, 1)).reshape(dims[3], D * 8)
    bt1r = bt1.reshape(1, D * 8)
    Wt2cat = jnp.transpose(Wt2, (0, 2, 1)).reshape(D * 8, 1)
    bt2r = bt2.reshape(D, 1)

    full = lambda shape: pl.BlockSpec(shape, lambda i: tuple(0 for _ in shape))
    grid = B // T
    out = pl.pallas_call(
        _body,
        grid=(grid,),
        in_specs=[
            pl.BlockSpec((T, IN), lambda i: (i, 0)),
            pl.BlockSpec((T, 1), lambda i: (i, 0)),
            full((L, D, DH + LP)),
            full((64, DH + LP)), full((1, 64)), full((1, 64)), full((1, 64)),
            full((1, 64)), full((1, 1)),
            full((64, DH + LP)), full((1, 64)), full((1, 64)), full((1, 64)),
            full((E, 64)), full((1, E)),
            full((E * R + dims[1], dims[0])), full((1, dims[1])), full((E * R, dims[1])),
            full((E * R + dims[2], dims[1])), full((1, dims[2])), full((E * R, dims[2])),
            full((E * R + dims[3], dims[2])), full((1, dims[3])), full((E * R, dims[3])),
            full((dims[3], D * 8)), full((1, D * 8)),
            full((D * 8, 1)), full((D, 1)),
        ],
        out_specs=pl.BlockSpec((T, 1), lambda i: (i, 0)),
        out_shape=jax.ShapeDtypeStruct((B, 1), jnp.float32),
        scratch_shapes=[pltpu.VMEM((L, D, E * R), jnp.float32)],
    )(x, dom2d, rin,
      Wi1, bi1.reshape(1, -1), gi.reshape(1, -1), bLNi.reshape(1, -1),
      Wi2, bi2.reshape(1, -1),
      Wq1, bq1.reshape(1, -1), gq.reshape(1, -1), bLNq.reshape(1, -1),
      Wq2, bq2.reshape(1, -1),
      WAs[0], brs[0], BCs[0],
      WAs[1], brs[1], BCs[1],
      WAs[2], brs[2], BCs[2],
      Wt1all, bt1r, Wt2cat, bt2r)
    return out


# speedup vs baseline: 8.4157x; 1.1588x over previous
import jax
import jax.numpy as jnp
from jax.experimental import pallas as pl
from jax.experimental.pallas import tpu as pltpu

B = 4096
NF = 26
ED = 64
IN = NF * ED
DIMS = [256, 128, 64]
E = 8
R = 16
L = 3
D = 4
KE = 2
KL = 2
DH = 64
LP = 32
SCALING = 1.0

T = 512


def _ln(x, g, b):
    m = jnp.mean(x, axis=-1, keepdims=True)
    v = jnp.mean((x - m) ** 2, axis=-1, keepdims=True)
    return (x - m) * jax.lax.rsqrt(v + 1e-5) * g + b


def _top2_softmax(v):
    m1 = jnp.max(v, axis=-1, keepdims=True)
    neg = jnp.float32(-3.0e38)
    v2 = jnp.where(v >= m1, neg, v)
    m2 = jnp.max(v2, axis=-1, keepdims=True)
    mask = v >= m2
    e = jnp.where(mask, jnp.exp(v - m1), 0.0)
    return e / jnp.sum(e, axis=-1, keepdims=True)


_DN = (((1,), (1,)), ((), ()))


def _body(x_ref, dom_ref, rin_ref,
          wi1_ref, bi1_ref, gi_ref, blni_ref, wi2_ref, bi2_ref,
          wq1_ref, bq1_ref, gq_ref, blnq_ref, wq2_ref, bq2_ref,
          w1_ref, b1_ref, a1_ref, lb1_ref,
          w2_ref, b2_ref, a2_ref, lb2_ref,
          w3_ref, b3_ref, a3_ref, lb3_ref,
          wt1_ref, bt1_ref, wt2_ref, bt2_ref,
          out_ref,
          stab_ref, wa1_ref, wa2_ref, wa3_ref, bc1_ref, bc2_ref, bc3_ref):
    i = pl.program_id(0)

    @pl.when(i == 0)
    def _prep():
        lane3 = jax.lax.broadcasted_iota(jnp.int32, (D, L), 1)
        rep = (jax.lax.broadcasted_iota(jnp.int32, (E, E * R), 0)
               == jax.lax.broadcasted_iota(jnp.int32, (E, E * R), 1) // R
               ).astype(jnp.float32)
        inter = jnp.zeros((D, L), jnp.float32)
        intra = []
        for l in range(L):
            r = rin_ref[l]
            zi = jax.lax.dot_general(r, wi1_ref[...], _DN, preferred_element_type=jnp.float32) + bi1_ref[...]
            hi = jnp.maximum(_ln(zi, gi_ref[...], blni_ref[...]), 0.0)
            il = jnp.sum(hi * wi2_ref[...], axis=-1, keepdims=True) + bi2_ref[...]
            inter = jnp.where(lane3 == l, il, inter)
            zq = jax.lax.dot_general(r, wq1_ref[...], _DN, preferred_element_type=jnp.float32) + bq1_ref[...]
            hq = jnp.maximum(_ln(zq, gq_ref[...], blnq_ref[...]), 0.0)
            ql = jax.lax.dot_general(hq, wq2_ref[...], _DN, preferred_element_type=jnp.float32) + bq2_ref[...]
            intra.append(_top2_softmax(ql))
        layer_w = _top2_softmax(inter)
        for l in range(L):
            lw = jnp.sum(jnp.where(lane3 == l, layer_w, 0.0), axis=-1, keepdims=True)
            gl = intra[l] * lw * SCALING
            stab_ref[l] = jnp.dot(gl, rep, preferred_element_type=jnp.float32)

        for ar, wr, war, lbr, bcr in (
                (a1_ref, w1_ref, wa1_ref, lb1_ref, bc1_ref),
                (a2_ref, w2_ref, wa2_ref, lb2_ref, bc2_ref),
                (a3_ref, w3_ref, wa3_ref, lb3_ref, bc3_ref)):
            war[:E * R, :] = ar[...].astype(jnp.bfloat16)
            war[E * R:, :] = wr[...].astype(jnp.bfloat16)
            for e in range(E):
                bcr[e * R:(e + 1) * R, :] = jnp.swapaxes(lbr[e], 0, 1).astype(jnp.bfloat16)

    dom = dom_ref[...]
    h = x_ref[...].astype(jnp.bfloat16)
    for l, (war, br, bcr) in enumerate(((wa1_ref, b1_ref, bc1_ref),
                                        (wa2_ref, b2_ref, bc2_ref),
                                        (wa3_ref, b3_ref, bc3_ref))):
        za = jax.lax.dot_general(h, war[...], _DN,
                                 preferred_element_type=jnp.float32)
        a = za[:, :E * R]
        z = za[:, E * R:] + br[...]
        st = stab_ref[l]
        s = jnp.zeros((T, E * R), jnp.float32)
        for d in range(D):
            s = jnp.where(dom == d, st[d:d + 1, :], s)
        lora = jax.lax.dot_general((a * s).astype(jnp.bfloat16), bcr[...],
                                   (((1,), (0,)), ((), ())),
                                   preferred_element_type=jnp.float32)
        hf = jnp.maximum(z + lora, 0.0)
        h = hf.astype(jnp.bfloat16)
    h = hf

    t = jnp.maximum(
        jax.lax.dot_general(h, wt1_ref[...], _DN, preferred_element_type=jnp.float32)
        + bt1_ref[...], 0.0)
    blk = jax.lax.broadcasted_iota(jnp.int32, (T, D * 8), 1) // 8
    tm = jnp.where(blk == dom, t, 0.0)
    o = jnp.sum(tm * wt2_ref[...], axis=-1, keepdims=True)
    ob = jnp.zeros((T, 1), jnp.float32)
    for d in range(D):
        ob = jnp.where(dom == d, bt2_ref[d:d + 1, :], ob)
    out_ref[...] = o + ob


def kernel(x, domain_id, W1, b1, W2, b2, W3, b3, loraA1, loraB1, loraA2, loraB2,
           loraA3, loraB3, dom_emb, layer_pos, Wi1, bi1, gi, bLNi, Wi2, bi2,
           Wq1, bq1, gq, bLNq, Wq2, bq2, Wt1, bt1, Wt2, bt2):
    dom2d = domain_id.astype(jnp.int32).reshape(B, 1)
    rin = jnp.concatenate([
        jnp.broadcast_to(dom_emb[None, :, :], (L, D, DH)),
        jnp.broadcast_to(layer_pos[:, None, :], (L, D, LP)),
    ], axis=-1)

    dims = [IN] + DIMS
    bf = jnp.bfloat16
    full = lambda shape: pl.BlockSpec(shape, lambda i: tuple(0 for _ in shape))
    grid = B // T
    out = pl.pallas_call(
        _body,
        grid=(grid,),
        in_specs=[
            pl.BlockSpec((T, IN), lambda i: (i, 0)),
            pl.BlockSpec((T, 1), lambda i: (i, 0)),
            full((L, D, DH + LP)),
            full((64, DH + LP)), full((1, 64)), full((1, 64)), full((1, 64)),
            full((1, 64)), full((1, 1)),
            full((64, DH + LP)), full((1, 64)), full((1, 64)), full((1, 64)),
            full((E, 64)), full((1, E)),
            full((dims[1], dims[0])), full((1, dims[1])), full((E * R, dims[0])), full((E, dims[1], R)),
            full((dims[2], dims[1])), full((1, dims[2])), full((E * R, dims[1])), full((E, dims[2], R)),
            full((dims[3], dims[2])), full((1, dims[3])), full((E * R, dims[2])), full((E, dims[3], R)),
            full((D * 8, dims[3])), full((1, D * 8)),
            full((1, D * 8)), full((D, 1)),
        ],
        out_specs=pl.BlockSpec((T, 1), lambda i: (i, 0)),
        out_shape=jax.ShapeDtypeStruct((B, 1), jnp.float32),
        scratch_shapes=[
            pltpu.VMEM((L, D, E * R), jnp.float32),
            pltpu.VMEM((E * R + dims[1], dims[0]), bf),
            pltpu.VMEM((E * R + dims[2], dims[1]), bf),
            pltpu.VMEM((E * R + dims[3], dims[2]), bf),
            pltpu.VMEM((E * R, dims[1]), bf),
            pltpu.VMEM((E * R, dims[2]), bf),
            pltpu.VMEM((E * R, dims[3]), bf),
        ],
    )(x, dom2d, rin,
      Wi1, bi1.reshape(1, -1), gi.reshape(1, -1), bLNi.reshape(1, -1),
      Wi2, bi2.reshape(1, -1),
      Wq1, bq1.reshape(1, -1), gq.reshape(1, -1), bLNq.reshape(1, -1),
      Wq2, bq2.reshape(1, -1),
      W1, b1.reshape(1, -1), loraA1.reshape(E * R, dims[0]), loraB1,
      W2, b2.reshape(1, -1), loraA2.reshape(E * R, dims[1]), loraB2,
      W3, b3.reshape(1, -1), loraA3.reshape(E * R, dims[2]), loraB3,
      Wt1.reshape(D * 8, dims[3]), bt1.reshape(1, D * 8),
      Wt2.reshape(1, D * 8), bt2.reshape(D, 1))
    return out


# speedup vs baseline: 9.6841x; 1.1507x over previous
import jax
import jax.numpy as jnp
from jax.experimental import pallas as pl
from jax.experimental.pallas import tpu as pltpu

B = 4096
NF = 26
ED = 64
IN = NF * ED
DIMS = [256, 128, 64]
E = 8
R = 16
L = 3
D = 4
KE = 2
KL = 2
DH = 64
LP = 32
SCALING = 1.0

T = 512


def _ln(x, g, b):
    m = jnp.mean(x, axis=-1, keepdims=True)
    v = jnp.mean((x - m) ** 2, axis=-1, keepdims=True)
    return (x - m) * jax.lax.rsqrt(v + 1e-5) * g + b


def _top2_softmax(v):
    m1 = jnp.max(v, axis=-1, keepdims=True)
    neg = jnp.float32(-3.0e38)
    v2 = jnp.where(v >= m1, neg, v)
    m2 = jnp.max(v2, axis=-1, keepdims=True)
    mask = v >= m2
    e = jnp.where(mask, jnp.exp(v - m1), 0.0)
    return e / jnp.sum(e, axis=-1, keepdims=True)


_DN = (((1,), (1,)), ((), ()))


def _body(x_ref, dom_ref, de_ref, lpos_ref,
          wi1_ref, bi1_ref, gi_ref, blni_ref, wi2_ref, bi2_ref,
          wq1_ref, bq1_ref, gq_ref, blnq_ref, wq2_ref, bq2_ref,
          w1_ref, b1_ref, a1_ref, lb1_ref,
          w2_ref, b2_ref, a2_ref, lb2_ref,
          w3_ref, b3_ref, a3_ref, lb3_ref,
          wt1_ref, bt1_ref, wt2_ref, bt2_ref,
          out_ref,
          stab_ref, wa1_ref, wa2_ref, wa3_ref, bc1_ref, bc2_ref, bc3_ref,
          trow_ref):
    i = pl.program_id(0)

    @pl.when(i == 0)
    def _prep():
        lane3 = jax.lax.broadcasted_iota(jnp.int32, (D, L), 1)
        rep = (jax.lax.broadcasted_iota(jnp.int32, (E, E * R), 0)
               == jax.lax.broadcasted_iota(jnp.int32, (E, E * R), 1) // R
               ).astype(jnp.float32)
        e4 = de_ref[...]
        wi1_e, wi1_l = wi1_ref[:, :DH], wi1_ref[:, DH:]
        wq1_e, wq1_l = wq1_ref[:, :DH], wq1_ref[:, DH:]
        zi_e = jax.lax.dot_general(e4, wi1_e, _DN, preferred_element_type=jnp.float32)
        zq_e = jax.lax.dot_general(e4, wq1_e, _DN, preferred_element_type=jnp.float32)
        inter = jnp.zeros((D, L), jnp.float32)
        intra = []
        for l in range(L):
            lp = lpos_ref[l:l + 1, :]
            zi = zi_e + jax.lax.dot_general(lp, wi1_l, _DN, preferred_element_type=jnp.float32) + bi1_ref[...]
            hi = jnp.maximum(_ln(zi, gi_ref[...], blni_ref[...]), 0.0)
            il = jnp.sum(hi * wi2_ref[...], axis=-1, keepdims=True) + bi2_ref[...]
            inter = jnp.where(lane3 == l, il, inter)
            zq = zq_e + jax.lax.dot_general(lp, wq1_l, _DN, preferred_element_type=jnp.float32) + bq1_ref[...]
            hq = jnp.maximum(_ln(zq, gq_ref[...], blnq_ref[...]), 0.0)
            ql = jax.lax.dot_general(hq, wq2_ref[...], _DN, preferred_element_type=jnp.float32) + bq2_ref[...]
            intra.append(_top2_softmax(ql))
        layer_w = _top2_softmax(inter)
        for l in range(L):
            lw = jnp.sum(jnp.where(lane3 == l, layer_w, 0.0), axis=-1, keepdims=True)
            gl = intra[l] * lw * SCALING
            stab_ref[l] = jnp.dot(gl, rep, preferred_element_type=jnp.float32)

        for ar, wr, war, lbr, bcr in (
                (a1_ref, w1_ref, wa1_ref, lb1_ref, bc1_ref),
                (a2_ref, w2_ref, wa2_ref, lb2_ref, bc2_ref),
                (a3_ref, w3_ref, wa3_ref, lb3_ref, bc3_ref)):
            war[:E * R, :] = ar[...].reshape(E * R, ar.shape[2]).astype(jnp.bfloat16)
            war[E * R:, :] = wr[...].astype(jnp.bfloat16)
            for e in range(E):
                bcr[e * R:(e + 1) * R, :] = jnp.swapaxes(lbr[e], 0, 1).astype(jnp.bfloat16)

        i0 = jax.lax.broadcasted_iota(jnp.int32, (8, D * 8), 0)
        i1 = jax.lax.broadcasted_iota(jnp.int32, (8, D * 8), 1)
        bt1row = jnp.zeros((1, D * 8), jnp.float32)
        wt2row = jnp.zeros((1, D * 8), jnp.float32)
        for d in range(D):
            pd = (i1 - 8 * d == i0).astype(jnp.float32)
            bt1row = bt1row + jnp.dot(bt1_ref[d:d + 1, :], pd, preferred_element_type=jnp.float32)
            wt2row = wt2row + jnp.dot(wt2_ref[d], pd, preferred_element_type=jnp.float32)
        trow_ref[0:1, :] = bt1row
        trow_ref[1:2, :] = wt2row

    dom = dom_ref[...]
    h = x_ref[...].astype(jnp.bfloat16)
    for l, (war, br, bcr) in enumerate(((wa1_ref, b1_ref, bc1_ref),
                                        (wa2_ref, b2_ref, bc2_ref),
                                        (wa3_ref, b3_ref, bc3_ref))):
        za = jax.lax.dot_general(h, war[...], _DN,
                                 preferred_element_type=jnp.float32)
        a = za[:, :E * R]
        z = za[:, E * R:] + br[...]
        st = stab_ref[l]
        s = jnp.zeros((T, E * R), jnp.float32)
        for d in range(D):
            s = jnp.where(dom == d, st[d:d + 1, :], s)
        lora = jax.lax.dot_general((a * s).astype(jnp.bfloat16), bcr[...],
                                   (((1,), (0,)), ((), ())),
                                   preferred_element_type=jnp.float32)
        hf = jnp.maximum(z + lora, 0.0)
        h = hf.astype(jnp.bfloat16)
    h = hf

    wt1 = wt1_ref[...].reshape(D * 8, DIMS[2])
    t = jnp.maximum(
        jax.lax.dot_general(h, wt1, _DN, preferred_element_type=jnp.float32)
        + trow_ref[0:1, :], 0.0)
    blk = jax.lax.broadcasted_iota(jnp.int32, (T, D * 8), 1) // 8
    tm = jnp.where(blk == dom, t, 0.0)
    o = jnp.sum(tm * trow_ref[1:2, :], axis=-1, keepdims=True)
    ob = jnp.zeros((T, 1), jnp.float32)
    for d in range(D):
        ob = jnp.where(dom == d, bt2_ref[d:d + 1, :], ob)
    out_ref[...] = o + ob


def kernel(x, domain_id, W1, b1, W2, b2, W3, b3, loraA1, loraB1, loraA2, loraB2,
           loraA3, loraB3, dom_emb, layer_pos, Wi1, bi1, gi, bLNi, Wi2, bi2,
           Wq1, bq1, gq, bLNq, Wq2, bq2, Wt1, bt1, Wt2, bt2):
    dom2d = domain_id.astype(jnp.int32).reshape(B, 1)

    dims = [IN] + DIMS
    bf = jnp.bfloat16
    full = lambda shape: pl.BlockSpec(shape, lambda i: tuple(0 for _ in shape))
    grid = B // T
    out = pl.pallas_call(
        _body,
        grid=(grid,),
        in_specs=[
            pl.BlockSpec((T, IN), lambda i: (i, 0)),
            pl.BlockSpec((T, 1), lambda i: (i, 0)),
            full((D, DH)), full((L, LP)),
            full((64, DH + LP)), full((1, 64)), full((1, 64)), full((1, 64)),
            full((1, 64)), full((1, 1)),
            full((64, DH + LP)), full((1, 64)), full((1, 64)), full((1, 64)),
            full((E, 64)), full((1, E)),
            full((dims[1], dims[0])), full((1, dims[1])), full((E, R, dims[0])), full((E, dims[1], R)),
            full((dims[2], dims[1])), full((1, dims[2])), full((E, R, dims[1])), full((E, dims[2], R)),
            full((dims[3], dims[2])), full((1, dims[3])), full((E, R, dims[2])), full((E, dims[3], R)),
            full((D, 8, dims[3])), full((D, 8)),
            full((D, 1, 8)), full((D, 1)),
        ],
        out_specs=pl.BlockSpec((T, 1), lambda i: (i, 0)),
        out_shape=jax.ShapeDtypeStruct((B, 1), jnp.float32),
        scratch_shapes=[
            pltpu.VMEM((L, D, E * R), jnp.float32),
            pltpu.VMEM((E * R + dims[1], dims[0]), bf),
            pltpu.VMEM((E * R + dims[2], dims[1]), bf),
            pltpu.VMEM((E * R + dims[3], dims[2]), bf),
            pltpu.VMEM((E * R, dims[1]), bf),
            pltpu.VMEM((E * R, dims[2]), bf),
            pltpu.VMEM((E * R, dims[3]), bf),
            pltpu.VMEM((2, D * 8), jnp.float32),
        ],
    )(x, dom2d, dom_emb, layer_pos,
      Wi1, bi1.reshape(1, -1), gi.reshape(1, -1), bLNi.reshape(1, -1),
      Wi2, bi2.reshape(1, -1),
      Wq1, bq1.reshape(1, -1), gq.reshape(1, -1), bLNq.reshape(1, -1),
      Wq2, bq2.reshape(1, -1),
      W1, b1.reshape(1, -1), loraA1, loraB1,
      W2, b2.reshape(1, -1), loraA2, loraB2,
      W3, b3.reshape(1, -1), loraA3, loraB3,
      Wt1, bt1, Wt2, bt2)
    return out


# speedup vs baseline: 9.6846x; 1.0001x over previous
import jax
import jax.numpy as jnp
from jax.experimental import pallas as pl
from jax.experimental.pallas import tpu as pltpu

B = 4096
NF = 26
ED = 64
IN = NF * ED
DIMS = [256, 128, 64]
E = 8
R = 16
L = 3
D = 4
KE = 2
KL = 2
DH = 64
LP = 32
SCALING = 1.0

T = 512


def _ln(x, g, b):
    m = jnp.mean(x, axis=-1, keepdims=True)
    v = jnp.mean((x - m) ** 2, axis=-1, keepdims=True)
    return (x - m) * jax.lax.rsqrt(v + 1e-5) * g + b


def _top2_softmax(v):
    m1 = jnp.max(v, axis=-1, keepdims=True)
    neg = jnp.float32(-3.0e38)
    v2 = jnp.where(v >= m1, neg, v)
    m2 = jnp.max(v2, axis=-1, keepdims=True)
    mask = v >= m2
    e = jnp.where(mask, jnp.exp(v - m1), 0.0)
    return e / jnp.sum(e, axis=-1, keepdims=True)


_DN = (((1,), (1,)), ((), ()))


def _body(x_ref, dom_ref, de_ref, lpos_ref,
          wi1_ref, bi1_ref, gi_ref, blni_ref, wi2_ref, bi2_ref,
          wq1_ref, bq1_ref, gq_ref, blnq_ref, wq2_ref, bq2_ref,
          w1_ref, b1_ref, a1_ref, lb1_ref,
          w2_ref, b2_ref, a2_ref, lb2_ref,
          w3_ref, b3_ref, a3_ref, lb3_ref,
          wt1_ref, bt1_ref, wt2_ref, bt2_ref,
          out_ref,
          stab_ref, wa1_ref, wa2_ref, wa3_ref, bc1_ref, bc2_ref, bc3_ref,
          trow_ref):
    i = pl.program_id(0)

    @pl.when(i == 0)
    def _prep():
        lane3 = jax.lax.broadcasted_iota(jnp.int32, (D, L), 1)
        rep = (jax.lax.broadcasted_iota(jnp.int32, (E, E * R), 0)
               == jax.lax.broadcasted_iota(jnp.int32, (E, E * R), 1) // R
               ).astype(jnp.float32)
        e4 = de_ref[...]
        wi1_e, wi1_l = wi1_ref[:, :DH], wi1_ref[:, DH:]
        wq1_e, wq1_l = wq1_ref[:, :DH], wq1_ref[:, DH:]
        zi_e = jax.lax.dot_general(e4, wi1_e, _DN, preferred_element_type=jnp.float32)
        zq_e = jax.lax.dot_general(e4, wq1_e, _DN, preferred_element_type=jnp.float32)
        inter = jnp.zeros((D, L), jnp.float32)
        intra = []
        for l in range(L):
            lp = lpos_ref[l:l + 1, :]
            zi = zi_e + jax.lax.dot_general(lp, wi1_l, _DN, preferred_element_type=jnp.float32) + bi1_ref[...].reshape(1, 64)
            hi = jnp.maximum(_ln(zi, gi_ref[...].reshape(1, 64), blni_ref[...].reshape(1, 64)), 0.0)
            il = jnp.sum(hi * wi2_ref[...], axis=-1, keepdims=True) + bi2_ref[...].reshape(1, 1)
            inter = jnp.where(lane3 == l, il, inter)
            zq = zq_e + jax.lax.dot_general(lp, wq1_l, _DN, preferred_element_type=jnp.float32) + bq1_ref[...].reshape(1, 64)
            hq = jnp.maximum(_ln(zq, gq_ref[...].reshape(1, 64), blnq_ref[...].reshape(1, 64)), 0.0)
            ql = jax.lax.dot_general(hq, wq2_ref[...], _DN, preferred_element_type=jnp.float32) + bq2_ref[...].reshape(1, E)
            intra.append(_top2_softmax(ql))
        layer_w = _top2_softmax(inter)
        for l in range(L):
            lw = jnp.sum(jnp.where(lane3 == l, layer_w, 0.0), axis=-1, keepdims=True)
            gl = intra[l] * lw * SCALING
            stab_ref[l] = jnp.dot(gl, rep, preferred_element_type=jnp.float32)

        for ar, wr, war, lbr, bcr in (
                (a1_ref, w1_ref, wa1_ref, lb1_ref, bc1_ref),
                (a2_ref, w2_ref, wa2_ref, lb2_ref, bc2_ref),
                (a3_ref, w3_ref, wa3_ref, lb3_ref, bc3_ref)):
            war[:E * R, :] = ar[...].reshape(E * R, ar.shape[2]).astype(jnp.bfloat16)
            war[E * R:, :] = wr[...].astype(jnp.bfloat16)
            for e in range(E):
                bcr[e * R:(e + 1) * R, :] = jnp.swapaxes(lbr[e], 0, 1).astype(jnp.bfloat16)

        i0 = jax.lax.broadcasted_iota(jnp.int32, (8, D * 8), 0)
        i1 = jax.lax.broadcasted_iota(jnp.int32, (8, D * 8), 1)
        bt1row = jnp.zeros((1, D * 8), jnp.float32)
        wt2row = jnp.zeros((1, D * 8), jnp.float32)
        for d in range(D):
            pd = (i1 - 8 * d == i0).astype(jnp.float32)
            bt1row = bt1row + jnp.dot(bt1_ref[d:d + 1, :], pd, preferred_element_type=jnp.float32)
            wt2row = wt2row + jnp.dot(wt2_ref[d], pd, preferred_element_type=jnp.float32)
        trow_ref[0:1, :] = bt1row
        trow_ref[1:2, :] = wt2row

    dom = dom_ref[...]
    h = x_ref[...].astype(jnp.bfloat16)
    for l, (war, br, bcr) in enumerate(((wa1_ref, b1_ref, bc1_ref),
                                        (wa2_ref, b2_ref, bc2_ref),
                                        (wa3_ref, b3_ref, bc3_ref))):
        za = jax.lax.dot_general(h, war[...], _DN,
                                 preferred_element_type=jnp.float32)
        a = za[:, :E * R]
        z = za[:, E * R:] + br[...].reshape(1, br.shape[0])
        st = stab_ref[l]
        s = jnp.zeros((T, E * R), jnp.float32)
        for d in range(D):
            s = jnp.where(dom == d, st[d:d + 1, :], s)
        lora = jax.lax.dot_general((a * s).astype(jnp.bfloat16), bcr[...],
                                   (((1,), (0,)), ((), ())),
                                   preferred_element_type=jnp.float32)
        hf = jnp.maximum(z + lora, 0.0)
        h = hf.astype(jnp.bfloat16)
    h = hf

    wt1 = wt1_ref[...].reshape(D * 8, DIMS[2])
    t = jnp.maximum(
        jax.lax.dot_general(h, wt1, _DN, preferred_element_type=jnp.float32)
        + trow_ref[0:1, :], 0.0)
    blk = jax.lax.broadcasted_iota(jnp.int32, (T, D * 8), 1) // 8
    tm = jnp.where(blk == dom, t, 0.0)
    o = jnp.sum(tm * trow_ref[1:2, :], axis=-1, keepdims=True)
    ob = jnp.zeros((T, 1), jnp.float32)
    for d in range(D):
        ob = jnp.where(dom == d, bt2_ref[d:d + 1, :], ob)
    out_ref[...] = o + ob


def kernel(x, domain_id, W1, b1, W2, b2, W3, b3, loraA1, loraB1, loraA2, loraB2,
           loraA3, loraB3, dom_emb, layer_pos, Wi1, bi1, gi, bLNi, Wi2, bi2,
           Wq1, bq1, gq, bLNq, Wq2, bq2, Wt1, bt1, Wt2, bt2):
    dom2d = domain_id.astype(jnp.int32).reshape(B, 1)

    dims = [IN] + DIMS
    bf = jnp.bfloat16
    full = lambda shape: pl.BlockSpec(shape, lambda i: tuple(0 for _ in shape))
    grid = B // T
    out = pl.pallas_call(
        _body,
        grid=(grid,),
        in_specs=[
            pl.BlockSpec((T, IN), lambda i: (i, 0)),
            pl.BlockSpec((T, 1), lambda i: (i, 0)),
            full((D, DH)), full((L, LP)),
            full((64, DH + LP)), full((64,)), full((64,)), full((64,)),
            full((1, 64)), full((1,)),
            full((64, DH + LP)), full((64,)), full((64,)), full((64,)),
            full((E, 64)), full((E,)),
            full((dims[1], dims[0])), full((dims[1],)), full((E, R, dims[0])), full((E, dims[1], R)),
            full((dims[2], dims[1])), full((dims[2],)), full((E, R, dims[1])), full((E, dims[2], R)),
            full((dims[3], dims[2])), full((dims[3],)), full((E, R, dims[2])), full((E, dims[3], R)),
            full((D, 8, dims[3])), full((D, 8)),
            full((D, 1, 8)), full((D, 1)),
        ],
        out_specs=pl.BlockSpec((T, 1), lambda i: (i, 0)),
        out_shape=jax.ShapeDtypeStruct((B, 1), jnp.float32),
        scratch_shapes=[
            pltpu.VMEM((L, D, E * R), jnp.float32),
            pltpu.VMEM((E * R + dims[1], dims[0]), bf),
            pltpu.VMEM((E * R + dims[2], dims[1]), bf),
            pltpu.VMEM((E * R + dims[3], dims[2]), bf),
            pltpu.VMEM((E * R, dims[1]), bf),
            pltpu.VMEM((E * R, dims[2]), bf),
            pltpu.VMEM((E * R, dims[3]), bf),
            pltpu.VMEM((2, D * 8), jnp.float32),
        ],
    )(x, dom2d, dom_emb, layer_pos,
      Wi1, bi1, gi, bLNi,
      Wi2, bi2,
      Wq1, bq1, gq, bLNq,
      Wq2, bq2,
      W1, b1, loraA1, loraB1,
      W2, b2, loraA2, loraB2,
      W3, b3, loraA3, loraB3,
      Wt1, bt1, Wt2, bt2)
    return out


# speedup vs baseline: 10.5462x; 1.0890x over previous
import jax
import jax.numpy as jnp
from jax.experimental import pallas as pl
from jax.experimental.pallas import tpu as pltpu

B = 4096
NF = 26
ED = 64
IN = NF * ED
DIMS = [256, 128, 64]
E = 8
R = 16
L = 3
D = 4
KE = 2
KL = 2
DH = 64
LP = 32
SCALING = 1.0

T = 1024


def _ln(x, g, b):
    m = jnp.mean(x, axis=-1, keepdims=True)
    v = jnp.mean((x - m) ** 2, axis=-1, keepdims=True)
    return (x - m) * jax.lax.rsqrt(v + 1e-5) * g + b


def _top2_softmax(v):
    m1 = jnp.max(v, axis=-1, keepdims=True)
    neg = jnp.float32(-3.0e38)
    v2 = jnp.where(v >= m1, neg, v)
    m2 = jnp.max(v2, axis=-1, keepdims=True)
    mask = v >= m2
    e = jnp.where(mask, jnp.exp(v - m1), 0.0)
    return e / jnp.sum(e, axis=-1, keepdims=True)


_DN = (((1,), (1,)), ((), ()))


def _body(x_ref, dom_ref, de_ref, lpos_ref,
          wi1_ref, bi1_ref, gi_ref, blni_ref, wi2_ref, bi2_ref,
          wq1_ref, bq1_ref, gq_ref, blnq_ref, wq2_ref, bq2_ref,
          w1_ref, b1_ref, a1_ref, lb1_ref,
          w2_ref, b2_ref, a2_ref, lb2_ref,
          w3_ref, b3_ref, a3_ref, lb3_ref,
          wt1_ref, bt1_ref, wt2_ref, bt2_ref,
          out_ref,
          stab_ref, wa1_ref, wa2_ref, wa3_ref, bc1_ref, bc2_ref, bc3_ref,
          trow_ref):
    i = pl.program_id(0)

    @pl.when(i == 0)
    def _prep():
        lane3 = jax.lax.broadcasted_iota(jnp.int32, (D, L), 1)
        rep = (jax.lax.broadcasted_iota(jnp.int32, (E, E * R), 0)
               == jax.lax.broadcasted_iota(jnp.int32, (E, E * R), 1) // R
               ).astype(jnp.float32)
        e4 = de_ref[...]
        wi1_e, wi1_l = wi1_ref[:, :DH], wi1_ref[:, DH:]
        wq1_e, wq1_l = wq1_ref[:, :DH], wq1_ref[:, DH:]
        zi_e = jax.lax.dot_general(e4, wi1_e, _DN, preferred_element_type=jnp.float32)
        zq_e = jax.lax.dot_general(e4, wq1_e, _DN, preferred_element_type=jnp.float32)
        inter = jnp.zeros((D, L), jnp.float32)
        intra = []
        for l in range(L):
            lp = lpos_ref[l:l + 1, :]
            zi = zi_e + jax.lax.dot_general(lp, wi1_l, _DN, preferred_element_type=jnp.float32) + bi1_ref[...].reshape(1, 64)
            hi = jnp.maximum(_ln(zi, gi_ref[...].reshape(1, 64), blni_ref[...].reshape(1, 64)), 0.0)
            il = jnp.sum(hi * wi2_ref[...], axis=-1, keepdims=True) + bi2_ref[...].reshape(1, 1)
            inter = jnp.where(lane3 == l, il, inter)
            zq = zq_e + jax.lax.dot_general(lp, wq1_l, _DN, preferred_element_type=jnp.float32) + bq1_ref[...].reshape(1, 64)
            hq = jnp.maximum(_ln(zq, gq_ref[...].reshape(1, 64), blnq_ref[...].reshape(1, 64)), 0.0)
            ql = jax.lax.dot_general(hq, wq2_ref[...], _DN, preferred_element_type=jnp.float32) + bq2_ref[...].reshape(1, E)
            intra.append(_top2_softmax(ql))
        layer_w = _top2_softmax(inter)
        for l in range(L):
            lw = jnp.sum(jnp.where(lane3 == l, layer_w, 0.0), axis=-1, keepdims=True)
            gl = intra[l] * lw * SCALING
            stab_ref[l] = jnp.dot(gl, rep, preferred_element_type=jnp.float32)

        for ar, wr, war, lbr, bcr in (
                (a1_ref, w1_ref, wa1_ref, lb1_ref, bc1_ref),
                (a2_ref, w2_ref, wa2_ref, lb2_ref, bc2_ref),
                (a3_ref, w3_ref, wa3_ref, lb3_ref, bc3_ref)):
            war[:E * R, :] = ar[...].reshape(E * R, ar.shape[2]).astype(jnp.bfloat16)
            war[E * R:, :] = wr[...].astype(jnp.bfloat16)
            for e in range(E):
                bcr[e * R:(e + 1) * R, :] = jnp.swapaxes(lbr[e], 0, 1).astype(jnp.bfloat16)

        i0 = jax.lax.broadcasted_iota(jnp.int32, (8, D * 8), 0)
        i1 = jax.lax.broadcasted_iota(jnp.int32, (8, D * 8), 1)
        bt1row = jnp.zeros((1, D * 8), jnp.float32)
        wt2row = jnp.zeros((1, D * 8), jnp.float32)
        for d in range(D):
            pd = (i1 - 8 * d == i0).astype(jnp.float32)
            bt1row = bt1row + jnp.dot(bt1_ref[d:d + 1, :], pd, preferred_element_type=jnp.float32)
            wt2row = wt2row + jnp.dot(wt2_ref[d], pd, preferred_element_type=jnp.float32)
        trow_ref[0:1, :] = bt1row
        trow_ref[1:2, :] = wt2row

    dom = dom_ref[...]
    h = x_ref[...].astype(jnp.bfloat16)
    for l, (war, br, bcr) in enumerate(((wa1_ref, b1_ref, bc1_ref),
                                        (wa2_ref, b2_ref, bc2_ref),
                                        (wa3_ref, b3_ref, bc3_ref))):
        za = jax.lax.dot_general(h, war[...], _DN,
                                 preferred_element_type=jnp.float32)
        a = za[:, :E * R]
        z = za[:, E * R:] + br[...].reshape(1, br.shape[0])
        st = stab_ref[l]
        s = jnp.zeros((T, E * R), jnp.float32)
        for d in range(D):
            s = jnp.where(dom == d, st[d:d + 1, :], s)
        lora = jax.lax.dot_general((a * s).astype(jnp.bfloat16), bcr[...],
                                   (((1,), (0,)), ((), ())),
                                   preferred_element_type=jnp.float32)
        hf = jnp.maximum(z + lora, 0.0)
        h = hf.astype(jnp.bfloat16)
    h = hf

    wt1 = wt1_ref[...].reshape(D * 8, DIMS[2])
    t = jnp.maximum(
        jax.lax.dot_general(h, wt1, _DN, preferred_element_type=jnp.float32)
        + trow_ref[0:1, :], 0.0)
    blk = jax.lax.broadcasted_iota(jnp.int32, (T, D * 8), 1) // 8
    tm = jnp.where(blk == dom, t, 0.0)
    o = jnp.sum(tm * trow_ref[1:2, :], axis=-1, keepdims=True)
    ob = jnp.zeros((T, 1), jnp.float32)
    for d in range(D):
        ob = jnp.where(dom == d, bt2_ref[d:d + 1, :], ob)
    out_ref[...] = o + ob


def kernel(x, domain_id, W1, b1, W2, b2, W3, b3, loraA1, loraB1, loraA2, loraB2,
           loraA3, loraB3, dom_emb, layer_pos, Wi1, bi1, gi, bLNi, Wi2, bi2,
           Wq1, bq1, gq, bLNq, Wq2, bq2, Wt1, bt1, Wt2, bt2):
    dom2d = domain_id.astype(jnp.int32).reshape(B, 1)

    dims = [IN] + DIMS
    bf = jnp.bfloat16
    full = lambda shape: pl.BlockSpec(shape, lambda i: tuple(0 for _ in shape))
    grid = B // T
    out = pl.pallas_call(
        _body,
        grid=(grid,),
        in_specs=[
            pl.BlockSpec((T, IN), lambda i: (i, 0)),
            pl.BlockSpec((T, 1), lambda i: (i, 0)),
            full((D, DH)), full((L, LP)),
            full((64, DH + LP)), full((64,)), full((64,)), full((64,)),
            full((1, 64)), full((1,)),
            full((64, DH + LP)), full((64,)), full((64,)), full((64,)),
            full((E, 64)), full((E,)),
            full((dims[1], dims[0])), full((dims[1],)), full((E, R, dims[0])), full((E, dims[1], R)),
            full((dims[2], dims[1])), full((dims[2],)), full((E, R, dims[1])), full((E, dims[2], R)),
            full((dims[3], dims[2])), full((dims[3],)), full((E, R, dims[2])), full((E, dims[3], R)),
            full((D, 8, dims[3])), full((D, 8)),
            full((D, 1, 8)), full((D, 1)),
        ],
        out_specs=pl.BlockSpec((T, 1), lambda i: (i, 0)),
        out_shape=jax.ShapeDtypeStruct((B, 1), jnp.float32),
        scratch_shapes=[
            pltpu.VMEM((L, D, E * R), jnp.float32),
            pltpu.VMEM((E * R + dims[1], dims[0]), bf),
            pltpu.VMEM((E * R + dims[2], dims[1]), bf),
            pltpu.VMEM((E * R + dims[3], dims[2]), bf),
            pltpu.VMEM((E * R, dims[1]), bf),
            pltpu.VMEM((E * R, dims[2]), bf),
            pltpu.VMEM((E * R, dims[3]), bf),
            pltpu.VMEM((2, D * 8), jnp.float32),
        ],
    )(x, dom2d, dom_emb, layer_pos,
      Wi1, bi1, gi, bLNi,
      Wi2, bi2,
      Wq1, bq1, gq, bLNq,
      Wq2, bq2,
      W1, b1, loraA1, loraB1,
      W2, b2, loraA2, loraB2,
      W3, b3, loraA3, loraB3,
      Wt1, bt1, Wt2, bt2)
    return out
